# Initial kernel scaffold; baseline (speedup 1.0000x reference)
import jax
import jax.numpy as jnp
from jax.experimental import pallas as pl


def kernel(x, table, W, b):
    raise NotImplementedError("write your pallas kernel here")


# speedup vs baseline: 2.7915x; 2.7915x over previous
import functools

import jax
import jax.numpy as jnp
from jax import lax
from jax.experimental import pallas as pl
from jax.experimental.pallas import tpu as pltpu
from jax.experimental.pallas import tpu_sc as plsc

NC = 2
NS = 16
LANES = 16


def _make_pool_kernel(B, CTX, V, D):
    NW = NC * NS
    assert B % (8 * NW) == 0
    b_per_w = B // NW
    n_vec = D // LANES
    inv_ctx = jnp.float32(1.0 / CTX)

    mesh = plsc.VectorSubcoreMesh(
        core_axis_name="c", subcore_axis_name="s",
        num_cores=NC, num_subcores=NS)

    @functools.partial(
        pl.kernel,
        mesh=mesh,
        out_type=jax.ShapeDtypeStruct((B, D), jnp.float32),
        scratch_types=[
            pltpu.VMEM((b_per_w, CTX), jnp.int32),
            pltpu.VMEM((CTX, D), jnp.float32),
            pltpu.VMEM((b_per_w, D), jnp.float32),
            pltpu.SemaphoreType.DMA,
        ],
    )
    def pool(x_hbm, table_hbm, out_hbm, idx_v, rows_v, pooled_v, sem):
        wid = lax.axis_index("s") * NC + lax.axis_index("c")
        base = wid * b_per_w
        pltpu.sync_copy(x_hbm.at[pl.ds(base, b_per_w)], idx_v)

        def row_body(bi, carry):
            pltpu.async_copy(table_hbm.at[idx_v.at[bi]], rows_v, sem).wait()
            accs = [rows_v[0, pl.ds(LANES * j, LANES)] for j in range(n_vec)]
            for r in range(1, CTX):
                accs = [accs[j] + rows_v[r, pl.ds(LANES * j, LANES)]
                        for j in range(n_vec)]
            for j in range(n_vec):
                pooled_v[bi, pl.ds(LANES * j, LANES)] = accs[j] * inv_ctx
            return carry

        lax.fori_loop(0, b_per_w, row_body, 0)
        pltpu.sync_copy(pooled_v, out_hbm.at[pl.ds(base, b_per_w)])

    return pool


def _make_dense_softmax(B, D, V, BB):
    def body(pooled_ref, w_ref, b_ref, out_ref):
        logits = jnp.dot(pooled_ref[...], w_ref[...],
                         preferred_element_type=jnp.float32) + b_ref[...]
        m = jnp.max(logits, axis=-1, keepdims=True)
        e = jnp.exp(logits - m)
        out_ref[...] = e / jnp.sum(e, axis=-1, keepdims=True)

    return pl.pallas_call(
        body,
        grid=(B // BB,),
        in_specs=[
            pl.BlockSpec((BB, D), lambda i: (i, 0)),
            pl.BlockSpec((D, V), lambda i: (0, 0)),
            pl.BlockSpec((1, V), lambda i: (0, 0)),
        ],
        out_specs=pl.BlockSpec((BB, V), lambda i: (i, 0)),
        out_shape=jax.ShapeDtypeStruct((B, V), jnp.float32),
    )


def kernel(x, table, W, b):
    B, CTX = x.shape
    V, D = table.shape
    x = x.astype(jnp.int32)
    pooled = _make_pool_kernel(B, CTX, V, D)(x, table)
    return _make_dense_softmax(B, D, V, 256)(pooled, W, b.reshape(1, V))


# speedup vs baseline: 3.1935x; 1.1440x over previous
import functools

import jax
import jax.numpy as jnp
from jax import lax
from jax.experimental import pallas as pl
from jax.experimental.pallas import tpu as pltpu
from jax.experimental.pallas import tpu_sc as plsc

NC = 2
NS = 16
LANES = 16


def _make_pool_kernel(B, CTX, V, D):
    NW = NC * NS
    NB = 4
    assert B % (8 * NW) == 0
    b_per_w = B // NW
    assert b_per_w % NB == 0
    n_groups = b_per_w // NB
    n_vec = D // LANES
    inv_ctx = jnp.float32(1.0 / CTX)

    mesh = plsc.VectorSubcoreMesh(
        core_axis_name="c", subcore_axis_name="s",
        num_cores=NC, num_subcores=NS)

    @functools.partial(
        pl.kernel,
        mesh=mesh,
        out_type=jax.ShapeDtypeStruct((B, D), jnp.float32),
        scratch_types=[
            pltpu.VMEM((b_per_w, CTX), jnp.int32),
            pltpu.VMEM((NB, CTX, D), jnp.float32),
            pltpu.VMEM((b_per_w, D), jnp.float32),
            [pltpu.SemaphoreType.DMA] * NB,
        ],
    )
    def pool(x_hbm, table_hbm, out_hbm, idx_v, rows_v, pooled_v, sems):
        wid = lax.axis_index("s") * NC + lax.axis_index("c")
        base = wid * b_per_w
        pltpu.sync_copy(x_hbm.at[pl.ds(base, b_per_w)], idx_v)

        def gather(s, b):
            return pltpu.make_async_copy(
                table_hbm.at[idx_v.at[s]], rows_v.at[b], sems[b])

        def reduce(s, b):
            accs = [rows_v[b, 0, pl.ds(LANES * j, LANES)] for j in range(n_vec)]
            for r in range(1, CTX):
                accs = [accs[j] + rows_v[b, r, pl.ds(LANES * j, LANES)]
                        for j in range(n_vec)]
            for j in range(n_vec):
                pooled_v[s, pl.ds(LANES * j, LANES)] = accs[j] * inv_ctx

        for b in range(NB):
            gather(b, b).start()

        def group_body(g, carry):
            for b in range(NB):
                s = g * NB + b
                gather(s, b).wait()
                reduce(s, b)
                gather(s + NB, b).start()
            return carry

        lax.fori_loop(0, n_groups - 1, group_body, 0)
        for b in range(NB):
            s = (n_groups - 1) * NB + b
            gather(s, b).wait()
            reduce(s, b)
        pltpu.sync_copy(pooled_v, out_hbm.at[pl.ds(base, b_per_w)])

    return pool


def _make_dense_softmax(B, D, V, BB):
    def body(pooled_ref, w_ref, b_ref, out_ref):
        logits = jnp.dot(pooled_ref[...], w_ref[...],
                         preferred_element_type=jnp.float32) + b_ref[...]
        m = jnp.max(logits, axis=-1, keepdims=True)
        e = jnp.exp(logits - m)
        out_ref[...] = e / jnp.sum(e, axis=-1, keepdims=True)

    return pl.pallas_call(
        body,
        grid=(B // BB,),
        in_specs=[
            pl.BlockSpec((BB, D), lambda i: (i, 0)),
            pl.BlockSpec((D, V), lambda i: (0, 0)),
            pl.BlockSpec((1, V), lambda i: (0, 0)),
        ],
        out_specs=pl.BlockSpec((BB, V), lambda i: (i, 0)),
        out_shape=jax.ShapeDtypeStruct((B, V), jnp.float32),
    )


def kernel(x, table, W, b):
    B, CTX = x.shape
    V, D = table.shape
    x = x.astype(jnp.int32)
    pooled = _make_pool_kernel(B, CTX, V, D)(x, table)
    return _make_dense_softmax(B, D, V, 256)(pooled, W, b.reshape(1, V))


# speedup vs baseline: 4.2439x; 1.3289x over previous
import functools

import jax
import jax.numpy as jnp
from jax import lax
from jax.experimental import pallas as pl
from jax.experimental.pallas import tpu as pltpu
from jax.experimental.pallas import tpu_sc as plsc

NC = 2
NS = 16
LANES = 16


def _make_pool_kernel(B, CTX, V, D):
    NW = NC * NS
    NB = 2
    assert B % (8 * NW) == 0
    b_per_w = B // NW
    assert b_per_w % NB == 0
    n_groups = b_per_w // NB
    n_vec = D // LANES
    inv_ctx = jnp.float32(1.0 / CTX)

    mesh = plsc.VectorSubcoreMesh(
        core_axis_name="c", subcore_axis_name="s",
        num_cores=NC, num_subcores=NS)

    @functools.partial(
        pl.kernel,
        mesh=mesh,
        out_type=jax.ShapeDtypeStruct((B, D), jnp.float32),
        scratch_types=[
            pltpu.VMEM((b_per_w, CTX), jnp.int32),
            pltpu.VMEM((NB, CTX, D), jnp.float32),
            pltpu.VMEM((b_per_w, D), jnp.float32),
            pltpu.VMEM_SHARED((V, D), jnp.float32),
            [pltpu.SemaphoreType.DMA] * NB,
        ],
    )
    def pool(x_hbm, table_hbm, out_hbm, idx_v, rows_v, pooled_v, tab_sh, sems):
        wid = lax.axis_index("s") * NC + lax.axis_index("c")
        base = wid * b_per_w
        sid = lax.axis_index("s")
        n_load = 10
        v_per_s = V // n_load
        assert V % n_load == 0 and v_per_s % 8 == 0

        @pl.when(sid < n_load)
        def _stage():
            pltpu.sync_copy(table_hbm.at[pl.ds(sid * v_per_s, v_per_s)],
                            tab_sh.at[pl.ds(sid * v_per_s, v_per_s)])

        pltpu.sync_copy(x_hbm.at[pl.ds(base, b_per_w)], idx_v)
        plsc.subcore_barrier()

        def gather(s, b):
            return pltpu.make_async_copy(
                tab_sh.at[idx_v.at[s]], rows_v.at[b], sems[b])

        UNROLL = 5
        assert CTX % UNROLL == 0

        def reduce(s, b):
            def acc_body(r0, accs):
                out = list(accs)
                for u in range(UNROLL):
                    r = r0 * UNROLL + u
                    out = [out[j] + rows_v[b, r, pl.ds(LANES * j, LANES)]
                           for j in range(n_vec)]
                return tuple(out)

            zero = jnp.zeros((LANES,), jnp.float32)
            accs = lax.fori_loop(0, CTX // UNROLL, acc_body,
                                 (zero,) * n_vec)
            for j in range(n_vec):
                pooled_v[s, pl.ds(LANES * j, LANES)] = accs[j] * inv_ctx

        for b in range(NB):
            gather(b, b).start()

        def group_body(g, carry):
            for b in range(NB):
                s = g * NB + b
                gather(s, b).wait()
                reduce(s, b)
                gather(s + NB, b).start()
            return carry

        lax.fori_loop(0, n_groups - 1, group_body, 0)
        for b in range(NB):
            s = (n_groups - 1) * NB + b
            gather(s, b).wait()
            reduce(s, b)
        pltpu.sync_copy(pooled_v, out_hbm.at[pl.ds(base, b_per_w)])

    return pool


def _make_dense_softmax(B, D, V, BB):
    def body(pooled_ref, w_ref, b_ref, out_ref):
        logits = jnp.dot(pooled_ref[...], w_ref[...],
                         preferred_element_type=jnp.float32) + b_ref[...]
        m = jnp.max(logits, axis=-1, keepdims=True)
        e = jnp.exp(logits - m)
        out_ref[...] = e / jnp.sum(e, axis=-1, keepdims=True)

    return pl.pallas_call(
        body,
        grid=(B // BB,),
        in_specs=[
            pl.BlockSpec((BB, D), lambda i: (i, 0)),
            pl.BlockSpec((D, V), lambda i: (0, 0)),
            pl.BlockSpec((1, V), lambda i: (0, 0)),
        ],
        out_specs=pl.BlockSpec((BB, V), lambda i: (i, 0)),
        out_shape=jax.ShapeDtypeStruct((B, V), jnp.float32),
    )


def kernel(x, table, W, b):
    B, CTX = x.shape
    V, D = table.shape
    x = x.astype(jnp.int32)
    pooled = _make_pool_kernel(B, CTX, V, D)(x, table)
    return _make_dense_softmax(B, D, V, 256)(pooled, W, b.reshape(1, V))


# speedup vs baseline: 4.4326x; 1.0445x over previous
import functools

import jax
import jax.numpy as jnp
from jax import lax
from jax.experimental import pallas as pl
from jax.experimental.pallas import tpu as pltpu
from jax.experimental.pallas import tpu_sc as plsc

NC = 2
NS = 16
LANES = 16


def _make_pool_kernel(B, CTX, V, D):
    NW = NC * NS
    NB = 2
    assert B % (8 * NW) == 0
    b_per_w = B // NW
    assert b_per_w % NB == 0
    n_groups = b_per_w // NB
    n_vec = D // LANES
    inv_ctx = jnp.float32(1.0 / CTX)

    mesh = plsc.VectorSubcoreMesh(
        core_axis_name="c", subcore_axis_name="s",
        num_cores=NC, num_subcores=NS)

    @functools.partial(
        pl.kernel,
        mesh=mesh,
        out_type=jax.ShapeDtypeStruct((B, D), jnp.float32),
        scratch_types=[
            pltpu.VMEM((b_per_w, CTX), jnp.int32),
            pltpu.VMEM((NB, CTX, D), jnp.float32),
            pltpu.VMEM((b_per_w, D), jnp.float32),
            pltpu.VMEM_SHARED((V, D), jnp.float32),
            [pltpu.SemaphoreType.DMA] * NB,
        ],
    )
    def pool(x_hbm, table_hbm, out_hbm, idx_v, rows_v, pooled_v, tab_sh, sems):
        wid = lax.axis_index("s") * NC + lax.axis_index("c")
        base = wid * b_per_w
        sid = lax.axis_index("s")
        n_load = 10
        v_per_s = V // n_load
        assert V % n_load == 0 and v_per_s % 8 == 0

        @pl.when(sid < n_load)
        def _stage():
            pltpu.sync_copy(table_hbm.at[pl.ds(sid * v_per_s, v_per_s)],
                            tab_sh.at[pl.ds(sid * v_per_s, v_per_s)])

        pltpu.sync_copy(x_hbm.at[pl.ds(base, b_per_w)], idx_v)
        plsc.subcore_barrier()

        def gather(s, b):
            return pltpu.make_async_copy(
                tab_sh.at[idx_v.at[s]], rows_v.at[b], sems[b])

        UNROLL = 5
        assert CTX % UNROLL == 0

        def reduce(s, b):
            def acc_body(r0, accs):
                out = list(accs)
                for u in range(UNROLL):
                    r = r0 * UNROLL + u
                    out = [out[j] + rows_v[b, r, pl.ds(LANES * j, LANES)]
                           for j in range(n_vec)]
                return tuple(out)

            zero = jnp.zeros((LANES,), jnp.float32)
            accs = lax.fori_loop(0, CTX // UNROLL, acc_body,
                                 (zero,) * n_vec)
            for j in range(n_vec):
                pooled_v[s, pl.ds(LANES * j, LANES)] = accs[j] * inv_ctx

        for b in range(NB):
            gather(b, b).start()

        def group_body(g, carry):
            for b in range(NB):
                s = g * NB + b
                gather(s, b).wait()
                reduce(s, b)
                gather(s + NB, b).start()
            return carry

        lax.fori_loop(0, n_groups - 1, group_body, 0)
        for b in range(NB):
            s = (n_groups - 1) * NB + b
            gather(s, b).wait()
            reduce(s, b)
        pltpu.sync_copy(pooled_v, out_hbm.at[pl.ds(base, b_per_w)])

    return pool


def _dense_softmax_chunk(B, D, V, BB, blk0, nblk, first):
    def body(pooled_ref, w_ref, b_ref, *rest):
        out_ref = rest[-1]
        logits = jnp.dot(pooled_ref[...], w_ref[...],
                         preferred_element_type=jnp.float32) + b_ref[...]
        m = jnp.max(logits, axis=-1, keepdims=True)
        e = jnp.exp(logits - m)
        out_ref[...] = e / jnp.sum(e, axis=-1, keepdims=True)

    in_specs = [
        pl.BlockSpec((BB, D), lambda i: (i, 0)),
        pl.BlockSpec((D, V), lambda i: (0, 0)),
        pl.BlockSpec((1, V), lambda i: (0, 0)),
    ]
    aliases = {}
    if not first:
        in_specs.append(pl.BlockSpec(memory_space=pl.ANY))
        aliases = {3: 0}
    return pl.pallas_call(
        body,
        grid=(nblk,),
        in_specs=in_specs,
        out_specs=pl.BlockSpec((BB, V), lambda i: (i + blk0, 0)),
        out_shape=jax.ShapeDtypeStruct((B, V), jnp.float32),
        input_output_aliases=aliases,
    )


def kernel(x, table, W, b):
    B, CTX = x.shape
    V, D = table.shape
    x = x.astype(jnp.int32)
    NCHUNK = 2
    BB = 256
    Bc = B // NCHUNK
    nblk = Bc // BB
    pool = _make_pool_kernel(Bc, CTX, V, D)
    b2 = b.reshape(1, V)
    pooled = [pool(x[c * Bc:(c + 1) * Bc], table) for c in range(NCHUNK)]
    out = _dense_softmax_chunk(B, D, V, BB, 0, nblk, True)(
        pooled[0], W, b2)
    for c in range(1, NCHUNK):
        out = _dense_softmax_chunk(B, D, V, BB, c * nblk, nblk, False)(
            pooled[c], W, b2, out)
    return out
